# baseline (speedup 1.0000x reference)
import functools

import jax
import jax.numpy as jnp
from jax import lax
from jax.experimental import pallas as pl
from jax.experimental.pallas import tpu as pltpu
from jax.experimental.pallas import tpu_sc as plsc

N_TOK = 8192
N_CODE = 8192
DIM = 256
TM = 512
TN = 2048
KBLKS = N_CODE // TN
IBLKS = N_TOK // TM
LOSS_SCALE = 0.25 / (N_TOK * DIM)


def _rne_bf16_f32(x):
    u = lax.bitcast_convert_type(x, jnp.uint32)
    r = (u + jnp.uint32(0x7FFF) + ((u >> 16) & jnp.uint32(1))) & jnp.uint32(
        0xFFFF0000)
    return lax.bitcast_convert_type(r, jnp.float32)


def _argmin_body(x_ref, w_ref, xsq_ref, esq_ref, idx_ref, loss_ref,
                 minv, mini, valv, acc):
    k = pl.program_id(0)
    i = pl.program_id(1)
    mm = lax.dot_general(
        x_ref[...], w_ref[...],
        dimension_numbers=(((1,), (1,)), ((), ())),
        preferred_element_type=jnp.float32,
    )
    dist = (xsq_ref[...] + esq_ref[...]) - 2.0 * mm
    bmin = jnp.min(dist, axis=1, keepdims=True)
    barg = (jnp.argmin(dist, axis=1).astype(jnp.int32).reshape(TM, 1)
            + k * TN)

    sl = pl.ds(i * TM, TM)
    first = k == 0
    prev_v = minv[sl, :]
    prev_i = mini[sl, :]
    prev_fv = valv[sl, :]
    take_new = jnp.logical_or(first, bmin < prev_v)
    new_v = jnp.where(take_new, _rne_bf16_f32(bmin), prev_v)
    new_i = jnp.where(take_new, barg, prev_i)
    new_fv = jnp.where(take_new, bmin, prev_fv)
    minv[sl, :] = new_v
    mini[sl, :] = new_i
    valv[sl, :] = new_fv

    @pl.when(k == KBLKS - 1)
    def _():
        idx_ref[...] = new_i
        s = jnp.sum(new_fv)
        prev = jnp.where(i == 0, 0.0, acc[0, 0])
        acc[0, 0] = prev + s

        @pl.when(i == IBLKS - 1)
        def _():
            loss_ref[0, 0] = acc[0, 0] * LOSS_SCALE


def _argmin_call(flat, weight, x_sq, e_sq_row):
    return pl.pallas_call(
        _argmin_body,
        grid=(KBLKS, IBLKS),
        in_specs=[
            pl.BlockSpec((TM, DIM), lambda k, i: (i, 0)),
            pl.BlockSpec((TN, DIM), lambda k, i: (k, 0)),
            pl.BlockSpec((TM, 1), lambda k, i: (i, 0)),
            pl.BlockSpec((1, TN), lambda k, i: (0, k)),
        ],
        out_specs=[
            pl.BlockSpec((TM, 1), lambda k, i: (i, 0)),
            pl.BlockSpec(memory_space=pltpu.SMEM),
        ],
        out_shape=[
            jax.ShapeDtypeStruct((N_TOK, 1), jnp.int32),
            jax.ShapeDtypeStruct((1, 1), jnp.float32),
        ],
        scratch_shapes=[
            pltpu.VMEM((N_TOK, 1), jnp.float32),
            pltpu.VMEM((N_TOK, 1), jnp.int32),
            pltpu.VMEM((N_TOK, 1), jnp.float32),
            pltpu.SMEM((1, 1), jnp.float32),
        ],
        compiler_params=pltpu.CompilerParams(
            dimension_semantics=("arbitrary", "arbitrary"),
        ),
    )(flat, weight, x_sq, e_sq_row)


def _onehot_body(idx_ref, out_ref):
    k = pl.program_id(1)
    cols = lax.broadcasted_iota(jnp.int32, (TM, TN), 1) + k * TN
    out_ref[...] = (idx_ref[...] == cols).astype(jnp.float32)


def _onehot_call(idx2):
    return pl.pallas_call(
        _onehot_body,
        grid=(IBLKS, KBLKS),
        in_specs=[pl.BlockSpec((TM, 1), lambda i, k: (i, 0))],
        out_specs=pl.BlockSpec((TM, TN), lambda i, k: (i, k)),
        out_shape=jax.ShapeDtypeStruct((N_TOK, N_CODE), jnp.float32),
        compiler_params=pltpu.CompilerParams(
            dimension_semantics=("parallel", "parallel"),
        ),
    )(idx2)


def _gather_rows(weight, idx_flat):
    info = plsc.get_sparse_core_info()
    nc, ns = info.num_cores, info.num_subcores
    nw = nc * ns
    b_per_w = N_TOK // nw
    mesh = plsc.VectorSubcoreMesh(core_axis_name="c", subcore_axis_name="s")

    @functools.partial(
        pl.kernel,
        mesh=mesh,
        out_type=jax.ShapeDtypeStruct((N_TOK, DIM), jnp.float32),
        scratch_types=[
            pltpu.VMEM((b_per_w,), jnp.int32),
            pltpu.VMEM((b_per_w, DIM), jnp.float32),
            pltpu.SemaphoreType.DMA,
        ],
    )
    def gather_k(w_hbm, idx_hbm, out_hbm, idx_v, rows_v, sem):
        wid = lax.axis_index("s") * nc + lax.axis_index("c")
        base = wid * b_per_w
        pltpu.sync_copy(idx_hbm.at[pl.ds(base, b_per_w)], idx_v)
        pltpu.async_copy(w_hbm.at[idx_v], rows_v, sem).wait()
        pltpu.sync_copy(rows_v, out_hbm.at[pl.ds(base, b_per_w)])

    return gather_k(weight, idx_flat)


def kernel(inputs, weight):
    B, N, D = inputs.shape
    flat = inputs.reshape(-1, D)
    x_sq = jnp.sum(flat ** 2, axis=1, keepdims=True)
    e_sq = jnp.sum(weight ** 2, axis=1)

    idx2, loss11 = _argmin_call(flat, weight, x_sq, e_sq.reshape(1, N_CODE))
    quant = _gather_rows(weight, idx2.reshape(N_TOK))
    enc = _onehot_call(idx2)
    return (
        quant.reshape(inputs.shape),
        enc.reshape(B, N, N_CODE),
        loss11[0, 0],
    )
